# Initial kernel scaffold; baseline (speedup 1.0000x reference)
import jax
import jax.numpy as jnp
from jax.experimental import pallas as pl


def kernel(sequence, embeddings):
    raise NotImplementedError("write your pallas kernel here")


# speedup vs baseline: 1.1031x; 1.1031x over previous
import functools

import jax
import jax.numpy as jnp
from jax import lax
from jax.experimental import pallas as pl
from jax.experimental.pallas import tpu as pltpu
from jax.experimental.pallas import tpu_sc as plsc

_NUM_CORES = 2
_NUM_SUBCORES = 16
_NW = _NUM_CORES * _NUM_SUBCORES

_G = 128
_K = 8
_CHUNK = _G * _K


def _make_gather(B: int, V: int, D: int):
    b_per_w = B // _NW
    n_chunks = b_per_w // _CHUNK
    mesh = plsc.VectorSubcoreMesh(core_axis_name="c", subcore_axis_name="s")

    @functools.partial(
        pl.kernel,
        out_type=jax.ShapeDtypeStruct((B, D), jnp.float32),
        mesh=mesh,
        scratch_types=[
            pltpu.VMEM((b_per_w,), jnp.int32),
            pltpu.VMEM((_CHUNK, D), jnp.float32),
            pltpu.SemaphoreType.DMA,
        ],
        compiler_params=pltpu.CompilerParams(use_tc_tiling_on_sc=False),
    )
    def gather_kernel(seq_hbm, table_hbm, out_hbm, idx_v, rows_v, sem):
        wid = lax.axis_index("s") * _NUM_CORES + lax.axis_index("c")
        base = wid * b_per_w
        pltpu.sync_copy(seq_hbm.at[pl.ds(base, b_per_w)], idx_v)

        def chunk_body(ci, carry):
            off = ci * _CHUNK
            copies = []
            for j in range(_K):
                copies.append(
                    pltpu.async_copy(
                        table_hbm.at[idx_v.at[pl.ds(off + j * _G, _G)]],
                        rows_v.at[pl.ds(j * _G, _G)],
                        sem,
                    )
                )
            for c in copies:
                c.wait()
            pltpu.sync_copy(rows_v, out_hbm.at[pl.ds(base + off, _CHUNK)])
            return carry

        lax.fori_loop(0, n_chunks, chunk_body, 0)

    return gather_kernel


def kernel(sequence, embeddings):
    Bq, H = sequence.shape
    V, D = embeddings.shape
    B = Bq * H
    seq_flat = sequence.reshape(B).astype(jnp.int32)
    out_flat = _make_gather(B, V, D)(seq_flat, embeddings)
    return out_flat.reshape(Bq, H, D)


# speedup vs baseline: 1.1090x; 1.0053x over previous
import functools

import jax
import jax.numpy as jnp
from jax import lax
from jax.experimental import pallas as pl
from jax.experimental.pallas import tpu as pltpu
from jax.experimental.pallas import tpu_sc as plsc

_NUM_CORES = 2
_NUM_SUBCORES = 16
_NW = _NUM_CORES * _NUM_SUBCORES

_G = 128
_K = 10
_CHUNK = _G * _K


def _make_gather(B: int, V: int, D: int):
    b_per_w = B // _NW
    n_chunks = b_per_w // _CHUNK
    assert n_chunks % 2 == 0 and n_chunks >= 4
    n_pairs = n_chunks // 2
    mesh = plsc.VectorSubcoreMesh(core_axis_name="c", subcore_axis_name="s")

    @functools.partial(
        pl.kernel,
        out_type=jax.ShapeDtypeStruct((B, D), jnp.float32),
        mesh=mesh,
        scratch_types=[
            pltpu.VMEM((b_per_w,), jnp.int32),
            pltpu.VMEM((_CHUNK, D), jnp.float32),
            pltpu.VMEM((_CHUNK, D), jnp.float32),
            pltpu.SemaphoreType.DMA,
            pltpu.SemaphoreType.DMA,
            pltpu.SemaphoreType.DMA,
            pltpu.SemaphoreType.DMA,
        ],
        compiler_params=pltpu.CompilerParams(use_tc_tiling_on_sc=False),
    )
    def gather_kernel(seq_hbm, table_hbm, out_hbm, idx_v, buf0, buf1,
                      gsem0, gsem1, wsem0, wsem1):
        wid = lax.axis_index("s") * _NUM_CORES + lax.axis_index("c")
        base = wid * b_per_w
        pltpu.sync_copy(seq_hbm.at[pl.ds(base, b_per_w)], idx_v)

        def fire(ci, buf, gsem):
            for j in range(_K):
                pltpu.async_copy(
                    table_hbm.at[idx_v.at[pl.ds(ci * _CHUNK + j * _G, _G)]],
                    buf.at[pl.ds(j * _G, _G)],
                    gsem,
                )

        def drain_gathers(buf, gsem):
            pltpu.make_async_copy(
                table_hbm.at[pl.ds(0, _CHUNK)], buf, gsem).wait()

        def start_wb(ci, buf, wsem):
            pltpu.async_copy(buf, out_hbm.at[pl.ds(base + ci * _CHUNK, _CHUNK)],
                             wsem)

        def drain_wb(buf, wsem):
            pltpu.make_async_copy(
                buf, out_hbm.at[pl.ds(base, _CHUNK)], wsem).wait()

        fire(0, buf0, gsem0)
        fire(1, buf1, gsem1)

        def body(g, carry):
            ci = 2 * g
            drain_gathers(buf0, gsem0)
            start_wb(ci, buf0, wsem0)
            drain_gathers(buf1, gsem1)
            start_wb(ci + 1, buf1, wsem1)
            drain_wb(buf0, wsem0)
            fire(ci + 2, buf0, gsem0)
            drain_wb(buf1, wsem1)
            fire(ci + 3, buf1, gsem1)
            return carry

        lax.fori_loop(0, n_pairs - 1, body, 0)

        ci = n_chunks - 2
        drain_gathers(buf0, gsem0)
        start_wb(ci, buf0, wsem0)
        drain_gathers(buf1, gsem1)
        start_wb(ci + 1, buf1, wsem1)
        drain_wb(buf0, wsem0)
        drain_wb(buf1, wsem1)

    return gather_kernel


def kernel(sequence, embeddings):
    Bq, H = sequence.shape
    V, D = embeddings.shape
    B = Bq * H
    seq_flat = sequence.reshape(B).astype(jnp.int32)
    out_flat = _make_gather(B, V, D)(seq_flat, embeddings)
    return out_flat.reshape(Bq, H, D)


# speedup vs baseline: 1.7943x; 1.6180x over previous
import functools

import jax
import jax.numpy as jnp
from jax import lax
from jax.experimental import pallas as pl
from jax.experimental.pallas import tpu as pltpu
from jax.experimental.pallas import tpu_sc as plsc

_NUM_CORES = 2
_NUM_SUBCORES = 16
_NW = _NUM_CORES * _NUM_SUBCORES

_NB = 16


def _make_gather(Bq: int, H: int, V: int, D: int):
    rows_per_w = Bq // _NW
    n_chunks = rows_per_w // _NB
    assert n_chunks % 2 == 0 and n_chunks >= 4
    n_pairs = n_chunks // 2
    mesh = plsc.VectorSubcoreMesh(core_axis_name="c", subcore_axis_name="s")

    @functools.partial(
        pl.kernel,
        out_type=jax.ShapeDtypeStruct((Bq, H, D), jnp.float32),
        mesh=mesh,
        scratch_types=[
            pltpu.VMEM((rows_per_w, H), jnp.int32),
            pltpu.VMEM((_NB, H, D), jnp.float32),
            pltpu.VMEM((_NB, H, D), jnp.float32),
            pltpu.SemaphoreType.DMA,
            pltpu.SemaphoreType.DMA,
            pltpu.SemaphoreType.DMA,
            pltpu.SemaphoreType.DMA,
        ],
        compiler_params=pltpu.CompilerParams(use_tc_tiling_on_sc=False),
    )
    def gather_kernel(seq_hbm, table_hbm, out_hbm, idx_v, buf0, buf1,
                      gsem0, gsem1, wsem0, wsem1):
        wid = lax.axis_index("s") * _NUM_CORES + lax.axis_index("c")
        base = wid * rows_per_w
        pltpu.sync_copy(seq_hbm.at[pl.ds(base, rows_per_w)], idx_v)

        def fire(ci, buf, gsem):
            for j in range(_NB):
                pltpu.async_copy(
                    table_hbm.at[idx_v.at[ci * _NB + j]],
                    buf.at[j],
                    gsem,
                )

        def drain_gathers(buf, gsem):
            pltpu.make_async_copy(
                out_hbm.at[pl.ds(0, _NB)], buf, gsem).wait()

        def start_wb(ci, buf, wsem):
            pltpu.async_copy(buf, out_hbm.at[pl.ds(base + ci * _NB, _NB)],
                             wsem)

        def drain_wb(buf, wsem):
            pltpu.make_async_copy(
                buf, out_hbm.at[pl.ds(base, _NB)], wsem).wait()

        fire(0, buf0, gsem0)
        fire(1, buf1, gsem1)

        def body(g, carry):
            ci = 2 * g
            drain_gathers(buf0, gsem0)
            start_wb(ci, buf0, wsem0)
            drain_gathers(buf1, gsem1)
            start_wb(ci + 1, buf1, wsem1)
            drain_wb(buf0, wsem0)
            fire(ci + 2, buf0, gsem0)
            drain_wb(buf1, wsem1)
            fire(ci + 3, buf1, gsem1)
            return carry

        lax.fori_loop(0, n_pairs - 1, body, 0)

        ci = n_chunks - 2
        drain_gathers(buf0, gsem0)
        start_wb(ci, buf0, wsem0)
        drain_gathers(buf1, gsem1)
        start_wb(ci + 1, buf1, wsem1)
        drain_wb(buf0, wsem0)
        drain_wb(buf1, wsem1)

    return gather_kernel


def kernel(sequence, embeddings):
    Bq, H = sequence.shape
    V, D = embeddings.shape
    return _make_gather(Bq, H, V, D)(sequence.astype(jnp.int32), embeddings)


# speedup vs baseline: 2.5279x; 1.4089x over previous
import functools

import jax
import jax.numpy as jnp
from jax import lax
from jax.experimental import pallas as pl
from jax.experimental.pallas import tpu as pltpu
from jax.experimental.pallas import tpu_sc as plsc

_NUM_CORES = 2
_NUM_SUBCORES = 16
_NW = _NUM_CORES * _NUM_SUBCORES

_NB = 16
_LANES = 128


def _make_gather(Bq: int, H: int, V: int, D: int, Hp: int):
    rows_per_w = Bq // _NW
    n_chunks = rows_per_w // _NB
    assert n_chunks % 2 == 0 and n_chunks >= 4
    n_pairs = n_chunks // 2
    mesh = plsc.VectorSubcoreMesh(core_axis_name="c", subcore_axis_name="s")

    @functools.partial(
        pl.kernel,
        out_type=jax.ShapeDtypeStruct((Bq * Hp, _LANES), jnp.float32),
        mesh=mesh,
        scratch_types=[
            pltpu.VMEM((rows_per_w, H), jnp.int32),
            pltpu.VMEM((_NB, H, D), jnp.float32),
            pltpu.VMEM((_NB, H, D), jnp.float32),
            pltpu.SemaphoreType.DMA,
            pltpu.SemaphoreType.DMA,
            pltpu.SemaphoreType.DMA,
            pltpu.SemaphoreType.DMA,
        ],
        compiler_params=pltpu.CompilerParams(use_tc_tiling_on_sc=False),
    )
    def gather_kernel(seq_hbm, table_hbm, out_hbm, idx_v, buf0, buf1,
                      gsem0, gsem1, wsem0, wsem1):
        wid = lax.axis_index("s") * _NUM_CORES + lax.axis_index("c")
        base = wid * rows_per_w
        pltpu.sync_copy(seq_hbm.at[pl.ds(base, rows_per_w)], idx_v)

        def fire(ci, buf, gsem):
            for j in range(_NB):
                pltpu.async_copy(
                    table_hbm.at[idx_v.at[ci * _NB + j]],
                    buf.at[j],
                    gsem,
                )

        def drain_gathers(buf, gsem):
            for j in range(_NB):
                pltpu.make_async_copy(
                    table_hbm.at[idx_v.at[j]], buf.at[j], gsem).wait()

        def start_wb(ci, buf, wsem):
            for j in range(_NB):
                bb = base + ci * _NB + j
                pltpu.async_copy(
                    buf.at[j],
                    out_hbm.at[pl.ds(bb * Hp, H), pl.ds(0, D)],
                    wsem,
                )

        def drain_wb(buf, wsem):
            for j in range(_NB):
                pltpu.make_async_copy(
                    buf.at[j],
                    out_hbm.at[pl.ds(0, H), pl.ds(0, D)], wsem).wait()

        fire(0, buf0, gsem0)
        fire(1, buf1, gsem1)

        def body(g, carry):
            ci = 2 * g
            drain_gathers(buf0, gsem0)
            start_wb(ci, buf0, wsem0)
            drain_gathers(buf1, gsem1)
            start_wb(ci + 1, buf1, wsem1)
            drain_wb(buf0, wsem0)
            fire(ci + 2, buf0, gsem0)
            drain_wb(buf1, wsem1)
            fire(ci + 3, buf1, gsem1)
            return carry

        lax.fori_loop(0, n_pairs - 1, body, 0)

        ci = n_chunks - 2
        drain_gathers(buf0, gsem0)
        start_wb(ci, buf0, wsem0)
        drain_gathers(buf1, gsem1)
        start_wb(ci + 1, buf1, wsem1)
        drain_wb(buf0, wsem0)
        drain_wb(buf1, wsem1)

    return gather_kernel


def kernel(sequence, embeddings):
    Bq, H = sequence.shape
    V, D = embeddings.shape
    Hp = (H + 7) // 8 * 8
    out128 = _make_gather(Bq, H, V, D, Hp)(sequence.astype(jnp.int32),
                                           embeddings)
    out3 = out128.reshape(Bq, Hp, _LANES)
    return out3[:, :H, :D]
